# baseline (speedup 1.0000x reference)
import functools

import jax
import jax.numpy as jnp
import numpy as np
from jax import lax
from jax.experimental import pallas as pl
from jax.experimental.pallas import tpu as pltpu
from jax.experimental.pallas import tpu_sc as plsc

NC = 2
NS = 16
NW = NC * NS
L = 16
K = 128

_SC_PARAMS = pltpu.CompilerParams(
    needs_layout_passes=False, use_tc_tiling_on_sc=False)


def _sc_mesh():
    return plsc.VectorSubcoreMesh(core_axis_name="c", subcore_axis_name="s")


def _eo_perm(F):
    p = []
    for g in range(0, F, 32):
        p.extend(range(g, g + 32, 2))
        p.extend(range(g + 1, g + 32, 2))
    return np.array(p, dtype=np.int64)


def _inv_perm(p):
    inv = np.empty_like(p)
    inv[p] = np.arange(len(p))
    return inv


def _bcast_lane(vec, l):
    idx = jnp.full((L, 1), l, jnp.int32)
    dnums = lax.GatherDimensionNumbers(
        offset_dims=(), collapsed_slice_dims=(0,), start_index_map=(0,))
    return lax.gather(vec, idx, dnums, (1,),
                      mode=lax.GatherScatterMode.PROMISE_IN_BOUNDS)


def _deg_kernel(NP, EP):
    T = EP // (NW * K)
    rpt = NP // NS

    @functools.partial(
        pl.kernel, mesh=_sc_mesh(),
        out_type=jax.ShapeDtypeStruct((NC, NP), jnp.float32),
        scratch_types=[
            pltpu.VMEM((T, K), jnp.int32),
            pltpu.VMEM((T, K), jnp.float32),
            pltpu.VMEM((rpt,), jnp.float32),
            pltpu.VMEM_SHARED((NP,), jnp.float32),
        ],
        compiler_params=_SC_PARAMS,
    )
    def deg_k(col_hbm, ew_hbm, out_hbm, col_all, ew_all, zbuf, acc_sh):
        c = lax.axis_index("c")
        s = lax.axis_index("s")
        w = c * NS + s
        pltpu.sync_copy(col_hbm.at[w], col_all)
        pltpu.sync_copy(ew_hbm.at[w], ew_all)

        def zb(i, carry):
            zbuf[pl.ds(i * L, L)] = jnp.zeros((L,), jnp.float32)
            return carry
        lax.fori_loop(0, rpt // L, zb, 0)
        pltpu.sync_copy(zbuf, acc_sh.at[pl.ds(s * rpt, rpt)])
        plsc.subcore_barrier()

        def chunk(t, carry):
            pltpu.sync_copy(ew_all.at[t], acc_sh.at[col_all.at[t]], add=True)
            return carry
        lax.fori_loop(0, T, chunk, 0)
        plsc.subcore_barrier()
        pltpu.sync_copy(acc_sh.at[pl.ds(s * rpt, rpt)],
                        out_hbm.at[c, pl.ds(s * rpt, rpt)])

    return deg_k


def _pick_piece(T, F, NP, NB, bf16):
    budget = 2_000_000
    bufw = (K * F * 3) // 2 if bf16 else K * F
    for P in range(T, 0, -1):
        if T % P or P % NB:
            continue
        if NS * (3 * P * K + NB * bufw) + NP * F <= budget:
            return P
    raise ValueError("no feasible staging piece size")


def _agg_kernel(NP, EP, F, feature_split, P, NB, bf16):
    ntile = NS if feature_split else NW
    T = EP // (ntile * K)
    QP = T // P
    assert T % P == 0 and P % NB == 0
    rpt = NP // NS
    ZC = rpt // K
    pdt = jnp.bfloat16 if bf16 else jnp.float32

    gather_bufs = [pltpu.VMEM((K, F), pdt) for _ in range(NB)]
    scaled_bufs = [pltpu.VMEM((K, F), jnp.float32)
                   for _ in range(NB)] if bf16 else []

    @functools.partial(
        pl.kernel, mesh=_sc_mesh(),
        out_type=jax.ShapeDtypeStruct((NC, NP, F), jnp.float32),
        scratch_types=[
            pltpu.VMEM((P, K), jnp.int32),
            pltpu.VMEM((P, K), jnp.int32),
            pltpu.VMEM((P, K), jnp.float32),
        ] + gather_bufs + scaled_bufs
          + [pltpu.VMEM_SHARED((NP, F), jnp.float32)]
          + [pltpu.SemaphoreType.DMA for _ in range(2 * NB)],
        compiler_params=_SC_PARAMS,
    )
    def agg_k(row_hbm, col_hbm, ew_hbm, p_hbm, out_hbm,
              row_all, col_all, ew_all, *bufs_acc_sems):
        rows = bufs_acc_sems[:NB]
        nsc = 2 * NB if bf16 else NB
        srows = bufs_acc_sems[NB:nsc] if bf16 else rows
        acc_sh = bufs_acc_sems[nsc]
        gsem = bufs_acc_sems[nsc + 1:nsc + 1 + NB]
        ssem = bufs_acc_sems[nsc + 1 + NB:]
        c = lax.axis_index("c")
        s = lax.axis_index("s")
        w = s if feature_split else c * NS + s

        def zb(i, carry):
            for j in range(F // L):
                srows[0][i, pl.ds(j * L, L)] = jnp.zeros((L,), jnp.float32)
            return carry
        lax.fori_loop(0, K, zb, 0)
        for z in range(ZC):
            pltpu.sync_copy(srows[0], acc_sh.at[pl.ds(s * rpt + z * K, K)])
        plsc.subcore_barrier()

        def scale(b, t):
            tv = jnp.full((L,), t, jnp.int32)

            def sbody(i, carry2):
                wv = plsc.load_gather(
                    ew_all, [tv, jnp.full((L,), i, jnp.int32)])
                if bf16:
                    wb = plsc.pack(wv, wv, format=plsc.PackFormat.INTERLEAVED)
                    for j in range(F // 32):
                        v = rows[b][i, pl.ds(j * 32, 32)]
                        prod = v * wb
                        pa, pb = plsc.unpack(
                            prod, format=plsc.PackFormat.INTERLEAVED)
                        srows[b][i, pl.ds(j * 32, L)] = pa
                        srows[b][i, pl.ds(j * 32 + L, L)] = pb
                else:
                    for j in range(F // L):
                        sl = (i, pl.ds(j * L, L))
                        rows[b][sl] = rows[b][sl] * wv
                return carry2

            if bf16:
                lax.fori_loop(0, K, sbody, 0, unroll=4)
            else:
                @plsc.parallel_loop(0, K, unroll=8)
                def _(i):
                    sbody(i, 0)

        def gstart(b, t):
            pltpu.async_copy(p_hbm.at[row_all.at[t]], rows[b], gsem[b])

        def gwait(b):
            pltpu.make_async_copy(
                p_hbm.at[row_all.at[0]], rows[b], gsem[b]).wait()

        def sstart(b, t):
            pltpu.async_copy(
                srows[b], acc_sh.at[col_all.at[t]], ssem[b], add=True)

        def swait(b):
            pltpu.make_async_copy(
                srows[b], acc_sh.at[col_all.at[0]], ssem[b]).wait()

        def piece(q, carry):
            pltpu.sync_copy(row_hbm.at[w, pl.ds(q * P, P)], row_all)
            pltpu.sync_copy(col_hbm.at[w, pl.ds(q * P, P)], col_all)
            pltpu.sync_copy(ew_hbm.at[w, pl.ds(q * P, P)], ew_all)
            if feature_split:
                offv = jnp.full((L,), c * NP, jnp.int32)

                def ob(i, carry2):
                    for j in range(K // L):
                        sl = (i, pl.ds(j * L, L))
                        row_all[sl] = row_all[sl] + offv
                    return carry2
                lax.fori_loop(0, P, ob, 0, unroll=4)

            for b in range(NB - 1):
                gstart(b, b)

            def rnd(r, carry2):
                for b in range(NB):
                    u = r * NB + b
                    gwait(b)
                    bp = (b - 1) % NB
                    if bf16:
                        @pl.when(r > 0)
                        def _():
                            swait(b)
                        scale(b, u)
                        gstart(bp, lax.rem(u + NB - 1, P))
                        sstart(b, u)
                    else:
                        scale(b, u)

                        @pl.when(u > 0)
                        def _():
                            swait(bp)
                        gstart(bp, lax.rem(u + NB - 1, P))
                        sstart(b, u)
                return carry2
            lax.fori_loop(0, P // NB, rnd, 0)
            for b in range(NB - 1):
                gwait(b)
            if bf16:
                for b in range(NB):
                    swait(b)
            else:
                swait(NB - 1)
            return carry
        lax.fori_loop(0, QP, piece, 0)
        plsc.subcore_barrier()
        for z in range(ZC):
            sl = pl.ds(s * rpt + z * K, K)
            pltpu.sync_copy(acc_sh.at[sl], out_hbm.at[c, sl])

    return agg_k


def _tc_norm_first(NP, D, H1, BLK):
    grid = NP // BLK
    FH = H1 // 2

    def body(dega, degb, x_ref, w_ref, dinv_ref, p_ref):
        d = dega[...] + degb[...] + 2.0
        di = jnp.where(d > 0, lax.rsqrt(d), 0.0)
        g = jnp.dot(x_ref[...], w_ref[...], preferred_element_type=jnp.float32)
        dinv_ref[...] = di
        p = di * g
        p_ref[0] = p[:, :FH]
        p_ref[1] = p[:, FH:]

    return pl.pallas_call(
        body,
        grid=(grid,),
        in_specs=[
            pl.BlockSpec((BLK, 1), lambda i: (i, 0)),
            pl.BlockSpec((BLK, 1), lambda i: (i, 0)),
            pl.BlockSpec((BLK, D), lambda i: (i, 0)),
            pl.BlockSpec((D, H1), lambda i: (0, 0)),
        ],
        out_specs=[
            pl.BlockSpec((BLK, 1), lambda i: (i, 0)),
            pl.BlockSpec((2, BLK, FH), lambda i: (0, i, 0)),
        ],
        out_shape=[
            jax.ShapeDtypeStruct((NP, 1), jnp.float32),
            jax.ShapeDtypeStruct((2, NP, FH), jnp.float32),
        ],
    )


def _tc_mid(NP, Fin, Fout, BLK, out_split):
    grid = NP // BLK
    FH = Fout // 2

    def body(acc_ref, p_ref, dinv_ref, b_ref, w_ref, pout_ref):
        di = dinv_ref[...]
        agg = jnp.concatenate([acc_ref[0], acc_ref[1]], axis=1)
        pin = jnp.concatenate([p_ref[0], p_ref[1]], axis=1)
        a = di * agg + (2.0 * di) * pin + b_ref[...]
        h = jnp.maximum(a, 0.0)
        gn = jnp.dot(h, w_ref[...], preferred_element_type=jnp.float32)
        p = di * gn
        if out_split:
            pout_ref[0] = p[:, :FH]
            pout_ref[1] = p[:, FH:]
        else:
            pout_ref[...] = p

    return pl.pallas_call(
        body,
        grid=(grid,),
        in_specs=[
            pl.BlockSpec((2, BLK, Fin // 2), lambda i: (0, i, 0)),
            pl.BlockSpec((2, BLK, Fin // 2), lambda i: (0, i, 0)),
            pl.BlockSpec((BLK, 1), lambda i: (i, 0)),
            pl.BlockSpec((1, Fin), lambda i: (0, 0)),
            pl.BlockSpec((Fin, Fout), lambda i: (0, 0)),
        ],
        out_specs=(
            pl.BlockSpec((2, BLK, FH), lambda i: (0, i, 0)) if out_split
            else pl.BlockSpec((BLK, Fout), lambda i: (i, 0))),
        out_shape=(
            jax.ShapeDtypeStruct((2, NP, FH), jnp.float32) if out_split
            else jax.ShapeDtypeStruct((NP, Fout), jnp.float32)),
    )


def _tc_final(NP, F3, C, BLK):
    grid = NP // BLK

    def body(acc_ref, p_ref, dinv_ref, b_ref, out_ref):
        di = dinv_ref[...]
        a = (di * (acc_ref[0] + acc_ref[1])
             + (2.0 * di) * p_ref[...] + b_ref[...])
        mask = lax.broadcasted_iota(jnp.int32, (BLK, F3), 1) < C
        am = jnp.where(mask, a, -1e30)
        m = jnp.max(am, axis=1, keepdims=True)
        z = am - m
        e = jnp.where(mask, jnp.exp(z), 0.0)
        ssum = jnp.sum(e, axis=1, keepdims=True)
        out_ref[...] = (z - jnp.log(ssum))[:, :C]

    return pl.pallas_call(
        body,
        grid=(grid,),
        in_specs=[
            pl.BlockSpec((2, BLK, F3), lambda i: (0, i, 0)),
            pl.BlockSpec((BLK, F3), lambda i: (i, 0)),
            pl.BlockSpec((BLK, 1), lambda i: (i, 0)),
            pl.BlockSpec((1, F3), lambda i: (0, 0)),
        ],
        out_specs=pl.BlockSpec((BLK, C), lambda i: (i, 0)),
        out_shape=jax.ShapeDtypeStruct((NP, C), jnp.float32),
    )


def kernel(x, edge_index, edge_attr, W1, b1, W2, b2, W3, b3):
    N, D = x.shape
    E = edge_index.shape[1]
    H1 = W1.shape[1]
    H2 = W2.shape[1]
    C = W3.shape[1]
    F3 = 16
    NP = -(-N // 2048) * 2048
    EP = -(-E // (NW * K * 2)) * (NW * K * 2)
    BLK = 2048

    row = edge_index[0]
    col = edge_index[1]
    pad = EP - E
    if pad:
        pidx = (jnp.arange(pad, dtype=jnp.int32) % N).astype(jnp.int32)
        row = jnp.concatenate([row, pidx])
        col = jnp.concatenate([col, pidx])
        ew = jnp.concatenate([edge_attr, jnp.zeros((pad,), edge_attr.dtype)])
    else:
        ew = edge_attr
    row32 = row.reshape(NW, -1, K)
    col32 = col.reshape(NW, -1, K)
    ew32 = ew.reshape(NW, -1, K)
    row16 = row.reshape(NS, -1, K)
    col16 = col.reshape(NS, -1, K)
    ew16 = ew.reshape(NS, -1, K)
    xp = jnp.pad(x, ((0, NP - N), (0, 0)))
    W3p = jnp.pad(W3, ((0, 0), (0, F3 - C)))
    b1r = b1[None, :]
    b2r = b2[None, :]
    b3r = jnp.pad(b3, (0, F3 - C))[None, :]

    deg = _deg_kernel(NP, EP)(col32, ew32)
    dega = deg[0][:, None]
    degb = deg[1][:, None]
    dinv, p1s = _tc_norm_first(NP, D, H1, BLK)(dega, degb, xp, W1)
    T16 = EP // (NS * K)
    T32 = EP // (NW * K)
    NB = 4 if T32 % 4 == 0 else 2
    p1f = p1s.reshape(2 * NP, H1 // 2)
    P1 = _pick_piece(T16, H1 // 2, NP, NB, False)
    acc1 = _agg_kernel(NP, EP, H1 // 2, True, P1, NB, False)(
        row16, col16, ew16, p1f)
    p2s = _tc_mid(NP, H1, H2, BLK, True)(acc1, p1s, dinv, b1r, W2)
    p2f = p2s.reshape(2 * NP, H2 // 2)
    P2 = _pick_piece(T16, H2 // 2, NP, NB, False)
    acc2 = _agg_kernel(NP, EP, H2 // 2, True, P2, NB, False)(
        row16, col16, ew16, p2f)
    p3 = _tc_mid(NP, H2, F3, BLK, False)(acc2, p2s, dinv, b2r, W3p)
    P3 = _pick_piece(T32, F3, NP, NB, False)
    acc3 = _agg_kernel(NP, EP, F3, False, P3, NB, False)(
        row32, col32, ew32, p3)
    out = _tc_final(NP, F3, C, BLK)(acc3, p3, dinv, b3r)
    return out[:N]


# speedup vs baseline: 1.0058x; 1.0058x over previous
import functools

import jax
import jax.numpy as jnp
import numpy as np
from jax import lax
from jax.experimental import pallas as pl
from jax.experimental.pallas import tpu as pltpu
from jax.experimental.pallas import tpu_sc as plsc

NC = 2
NS = 16
NW = NC * NS
L = 16
K = 128

_SC_PARAMS = pltpu.CompilerParams(
    needs_layout_passes=False, use_tc_tiling_on_sc=False)


def _sc_mesh():
    return plsc.VectorSubcoreMesh(core_axis_name="c", subcore_axis_name="s")


def _eo_perm(F):
    p = []
    for g in range(0, F, 32):
        p.extend(range(g, g + 32, 2))
        p.extend(range(g + 1, g + 32, 2))
    return np.array(p, dtype=np.int64)


def _inv_perm(p):
    inv = np.empty_like(p)
    inv[p] = np.arange(len(p))
    return inv


def _bcast_lane(vec, l):
    idx = jnp.full((L, 1), l, jnp.int32)
    dnums = lax.GatherDimensionNumbers(
        offset_dims=(), collapsed_slice_dims=(0,), start_index_map=(0,))
    return lax.gather(vec, idx, dnums, (1,),
                      mode=lax.GatherScatterMode.PROMISE_IN_BOUNDS)


def _deg_kernel(NP, EP):
    T = EP // (NW * K)
    rpt = NP // NS

    @functools.partial(
        pl.kernel, mesh=_sc_mesh(),
        out_type=jax.ShapeDtypeStruct((NC, NP), jnp.float32),
        scratch_types=[
            pltpu.VMEM((T, 2, K), jnp.int32),
            pltpu.VMEM((T, K), jnp.float32),
            pltpu.VMEM((rpt,), jnp.float32),
            pltpu.VMEM_SHARED((NP,), jnp.float32),
        ],
        compiler_params=_SC_PARAMS,
    )
    def deg_k(ei_hbm, ew_hbm, out_hbm, ei_all, ew_all, zbuf, acc_sh):
        c = lax.axis_index("c")
        s = lax.axis_index("s")
        w = c * NS + s
        pltpu.sync_copy(ei_hbm.at[w], ei_all)
        pltpu.sync_copy(ew_hbm.at[w], ew_all)

        def zb(i, carry):
            zbuf[pl.ds(i * L, L)] = jnp.zeros((L,), jnp.float32)
            return carry
        lax.fori_loop(0, rpt // L, zb, 0)
        pltpu.sync_copy(zbuf, acc_sh.at[pl.ds(s * rpt, rpt)])
        plsc.subcore_barrier()

        def chunk(t, carry):
            pltpu.sync_copy(ew_all.at[t], acc_sh.at[ei_all.at[t, 1]], add=True)
            return carry
        lax.fori_loop(0, T, chunk, 0)
        plsc.subcore_barrier()
        pltpu.sync_copy(acc_sh.at[pl.ds(s * rpt, rpt)],
                        out_hbm.at[c, pl.ds(s * rpt, rpt)])

    return deg_k


def _pick_piece(T, F, NP, NB, bf16):
    budget = 2_000_000
    bufw = (K * F * 3) // 2 if bf16 else K * F
    for P in range(T, 0, -1):
        if T % P or P % NB:
            continue
        if NS * (3 * P * K + NB * bufw) + NP * F <= budget:
            return P
    raise ValueError("no feasible staging piece size")


def _agg_kernel(NP, EP, F, feature_split, P, NB, bf16):
    ntile = NS if feature_split else NW
    T = EP // (ntile * K)
    QP = T // P
    assert T % P == 0 and P % NB == 0
    rpt = NP // NS
    ZC = rpt // K
    pdt = jnp.bfloat16 if bf16 else jnp.float32

    gather_bufs = [pltpu.VMEM((K, F), pdt) for _ in range(NB)]
    scaled_bufs = [pltpu.VMEM((K, F), jnp.float32)
                   for _ in range(NB)] if bf16 else []

    @functools.partial(
        pl.kernel, mesh=_sc_mesh(),
        out_type=jax.ShapeDtypeStruct((NC, NP, F), jnp.float32),
        scratch_types=[
            pltpu.VMEM((P, 2, K), jnp.int32),
            pltpu.VMEM((P, K), jnp.float32),
        ] + gather_bufs + scaled_bufs
          + [pltpu.VMEM_SHARED((NP, F), jnp.float32)]
          + [pltpu.SemaphoreType.DMA for _ in range(2 * NB)],
        compiler_params=_SC_PARAMS,
    )
    def agg_k(ei_hbm, ew_hbm, p_hbm, out_hbm,
              ei_all, ew_all, *bufs_acc_sems):
        rows = bufs_acc_sems[:NB]
        nsc = 2 * NB if bf16 else NB
        srows = bufs_acc_sems[NB:nsc] if bf16 else rows
        acc_sh = bufs_acc_sems[nsc]
        gsem = bufs_acc_sems[nsc + 1:nsc + 1 + NB]
        ssem = bufs_acc_sems[nsc + 1 + NB:]
        c = lax.axis_index("c")
        s = lax.axis_index("s")
        w = s if feature_split else c * NS + s

        def zb(i, carry):
            for j in range(F // L):
                srows[0][i, pl.ds(j * L, L)] = jnp.zeros((L,), jnp.float32)
            return carry
        lax.fori_loop(0, K, zb, 0)
        for z in range(ZC):
            pltpu.sync_copy(srows[0], acc_sh.at[pl.ds(s * rpt + z * K, K)])
        plsc.subcore_barrier()

        def scale(b, t):
            tv = jnp.full((L,), t, jnp.int32)

            def sbody(i, carry2):
                wv = plsc.load_gather(
                    ew_all, [tv, jnp.full((L,), i, jnp.int32)])
                if bf16:
                    wb = plsc.pack(wv, wv, format=plsc.PackFormat.INTERLEAVED)
                    for j in range(F // 32):
                        v = rows[b][i, pl.ds(j * 32, 32)]
                        prod = v * wb
                        pa, pb = plsc.unpack(
                            prod, format=plsc.PackFormat.INTERLEAVED)
                        srows[b][i, pl.ds(j * 32, L)] = pa
                        srows[b][i, pl.ds(j * 32 + L, L)] = pb
                else:
                    for j in range(F // L):
                        sl = (i, pl.ds(j * L, L))
                        rows[b][sl] = rows[b][sl] * wv
                return carry2

            if bf16:
                lax.fori_loop(0, K, sbody, 0, unroll=4)
            else:
                @plsc.parallel_loop(0, K, unroll=8)
                def _(i):
                    sbody(i, 0)

        def gstart(b, t):
            pltpu.async_copy(p_hbm.at[ei_all.at[t, 0]], rows[b], gsem[b])

        def gwait(b):
            pltpu.make_async_copy(
                p_hbm.at[ei_all.at[0, 0]], rows[b], gsem[b]).wait()

        def sstart(b, t):
            pltpu.async_copy(
                srows[b], acc_sh.at[ei_all.at[t, 1]], ssem[b], add=True)

        def swait(b):
            pltpu.make_async_copy(
                srows[b], acc_sh.at[ei_all.at[0, 1]], ssem[b]).wait()

        def piece(q, carry):
            pltpu.sync_copy(ei_hbm.at[w, pl.ds(q * P, P)], ei_all)
            pltpu.sync_copy(ew_hbm.at[w, pl.ds(q * P, P)], ew_all)
            if feature_split:
                offv = jnp.full((L,), c * NP, jnp.int32)

                def ob(i, carry2):
                    for j in range(K // L):
                        sl = (i, 0, pl.ds(j * L, L))
                        ei_all[sl] = ei_all[sl] + offv
                    return carry2
                lax.fori_loop(0, P, ob, 0, unroll=4)

            for b in range(NB - 1):
                gstart(b, b)

            def rnd(r, carry2):
                for b in range(NB):
                    u = r * NB + b
                    gwait(b)
                    bp = (b - 1) % NB
                    if bf16:
                        @pl.when(r > 0)
                        def _():
                            swait(b)
                        scale(b, u)
                        gstart(bp, lax.rem(u + NB - 1, P))
                        sstart(b, u)
                    else:
                        scale(b, u)

                        @pl.when(u > 0)
                        def _():
                            swait(bp)
                        gstart(bp, lax.rem(u + NB - 1, P))
                        sstart(b, u)
                return carry2
            lax.fori_loop(0, P // NB, rnd, 0)
            for b in range(NB - 1):
                gwait(b)
            if bf16:
                for b in range(NB):
                    swait(b)
            else:
                swait(NB - 1)
            return carry
        lax.fori_loop(0, QP, piece, 0)
        plsc.subcore_barrier()
        for z in range(ZC):
            sl = pl.ds(s * rpt + z * K, K)
            pltpu.sync_copy(acc_sh.at[sl], out_hbm.at[c, sl])

    return agg_k


def _tc_norm_first(NP, D, H1, BLK):
    grid = NP // BLK
    FH = H1 // 2

    def body(dega, degb, x_ref, w_ref, dinv_ref, p_ref):
        d = dega[...] + degb[...] + 2.0
        di = jnp.where(d > 0, lax.rsqrt(d), 0.0)
        g = jnp.dot(x_ref[...], w_ref[...], preferred_element_type=jnp.float32)
        dinv_ref[...] = di
        p = di * g
        p_ref[0] = p[:, :FH]
        p_ref[1] = p[:, FH:]

    return pl.pallas_call(
        body,
        grid=(grid,),
        in_specs=[
            pl.BlockSpec((BLK, 1), lambda i: (i, 0)),
            pl.BlockSpec((BLK, 1), lambda i: (i, 0)),
            pl.BlockSpec((BLK, D), lambda i: (i, 0)),
            pl.BlockSpec((D, H1), lambda i: (0, 0)),
        ],
        out_specs=[
            pl.BlockSpec((BLK, 1), lambda i: (i, 0)),
            pl.BlockSpec((2, BLK, FH), lambda i: (0, i, 0)),
        ],
        out_shape=[
            jax.ShapeDtypeStruct((NP, 1), jnp.float32),
            jax.ShapeDtypeStruct((2, NP, FH), jnp.float32),
        ],
    )


def _tc_mid(NP, Fin, Fout, BLK, out_split):
    grid = NP // BLK
    FH = Fout // 2

    def body(acc_ref, p_ref, dinv_ref, b_ref, w_ref, pout_ref):
        di = dinv_ref[...]
        agg = jnp.concatenate([acc_ref[0], acc_ref[1]], axis=1)
        pin = jnp.concatenate([p_ref[0], p_ref[1]], axis=1)
        a = di * agg + (2.0 * di) * pin + b_ref[...]
        h = jnp.maximum(a, 0.0)
        gn = jnp.dot(h, w_ref[...], preferred_element_type=jnp.float32)
        p = di * gn
        if out_split:
            pout_ref[0] = p[:, :FH]
            pout_ref[1] = p[:, FH:]
        else:
            pout_ref[...] = p

    return pl.pallas_call(
        body,
        grid=(grid,),
        in_specs=[
            pl.BlockSpec((2, BLK, Fin // 2), lambda i: (0, i, 0)),
            pl.BlockSpec((2, BLK, Fin // 2), lambda i: (0, i, 0)),
            pl.BlockSpec((BLK, 1), lambda i: (i, 0)),
            pl.BlockSpec((1, Fin), lambda i: (0, 0)),
            pl.BlockSpec((Fin, Fout), lambda i: (0, 0)),
        ],
        out_specs=(
            pl.BlockSpec((2, BLK, FH), lambda i: (0, i, 0)) if out_split
            else pl.BlockSpec((BLK, Fout), lambda i: (i, 0))),
        out_shape=(
            jax.ShapeDtypeStruct((2, NP, FH), jnp.float32) if out_split
            else jax.ShapeDtypeStruct((NP, Fout), jnp.float32)),
    )


def _tc_final(NP, F3, C, BLK):
    grid = NP // BLK

    def body(acc_ref, p_ref, dinv_ref, b_ref, out_ref):
        di = dinv_ref[...]
        a = (di * (acc_ref[0] + acc_ref[1])
             + (2.0 * di) * p_ref[...] + b_ref[...])
        mask = lax.broadcasted_iota(jnp.int32, (BLK, F3), 1) < C
        am = jnp.where(mask, a, -1e30)
        m = jnp.max(am, axis=1, keepdims=True)
        z = am - m
        e = jnp.where(mask, jnp.exp(z), 0.0)
        ssum = jnp.sum(e, axis=1, keepdims=True)
        out_ref[...] = (z - jnp.log(ssum))[:, :C]

    return pl.pallas_call(
        body,
        grid=(grid,),
        in_specs=[
            pl.BlockSpec((2, BLK, F3), lambda i: (0, i, 0)),
            pl.BlockSpec((BLK, F3), lambda i: (i, 0)),
            pl.BlockSpec((BLK, 1), lambda i: (i, 0)),
            pl.BlockSpec((1, F3), lambda i: (0, 0)),
        ],
        out_specs=pl.BlockSpec((BLK, C), lambda i: (i, 0)),
        out_shape=jax.ShapeDtypeStruct((NP, C), jnp.float32),
    )


def kernel(x, edge_index, edge_attr, W1, b1, W2, b2, W3, b3):
    N, D = x.shape
    E = edge_index.shape[1]
    H1 = W1.shape[1]
    H2 = W2.shape[1]
    C = W3.shape[1]
    F3 = 16
    NP = -(-N // 2048) * 2048
    EP = -(-E // (NW * K * 2)) * (NW * K * 2)
    BLK = 2048

    if E % K:
        ek = -(-E // K) * K - E
        fill = (jnp.arange(ek, dtype=jnp.int32) % N).astype(jnp.int32)
        edge_index = jnp.concatenate(
            [edge_index, jnp.broadcast_to(fill, (2, ek))], axis=1)
        edge_attr = jnp.concatenate(
            [edge_attr, jnp.zeros((ek,), edge_attr.dtype)])
        E += ek
    ei3 = jnp.transpose(edge_index.reshape(2, E // K, K), (1, 0, 2))
    pad = EP - E
    if pad:
        pidx = (jnp.arange(pad, dtype=jnp.int32) % N).astype(jnp.int32)
        pad3 = jnp.broadcast_to(
            pidx.reshape(pad // K, 1, K), (pad // K, 2, K))
        ei3 = jnp.concatenate([ei3, pad3], axis=0)
        ew = jnp.concatenate([edge_attr, jnp.zeros((pad,), edge_attr.dtype)])
    else:
        ew = edge_attr
    ei32 = ei3.reshape(NW, -1, 2, K)
    ew32 = ew.reshape(NW, -1, K)
    ei16 = ei3.reshape(NS, -1, 2, K)
    ew16 = ew.reshape(NS, -1, K)
    xp = jnp.pad(x, ((0, NP - N), (0, 0)))
    W3p = jnp.pad(W3, ((0, 0), (0, F3 - C)))
    b1r = b1[None, :]
    b2r = b2[None, :]
    b3r = jnp.pad(b3, (0, F3 - C))[None, :]

    deg = _deg_kernel(NP, EP)(ei32, ew32)
    dega = deg[0][:, None]
    degb = deg[1][:, None]
    dinv, p1s = _tc_norm_first(NP, D, H1, BLK)(dega, degb, xp, W1)
    T16 = EP // (NS * K)
    T32 = EP // (NW * K)
    NB = 4 if T32 % 4 == 0 else 2
    p1f = p1s.reshape(2 * NP, H1 // 2)
    P1 = _pick_piece(T16, H1 // 2, NP, NB, False)
    acc1 = _agg_kernel(NP, EP, H1 // 2, True, P1, NB, False)(
        ei16, ew16, p1f)
    p2s = _tc_mid(NP, H1, H2, BLK, True)(acc1, p1s, dinv, b1r, W2)
    p2f = p2s.reshape(2 * NP, H2 // 2)
    P2 = _pick_piece(T16, H2 // 2, NP, NB, False)
    acc2 = _agg_kernel(NP, EP, H2 // 2, True, P2, NB, False)(
        ei16, ew16, p2f)
    p3 = _tc_mid(NP, H2, F3, BLK, False)(acc2, p2s, dinv, b2r, W3p)
    P3 = _pick_piece(T32, F3, NP, NB, False)
    acc3 = _agg_kernel(NP, EP, F3, False, P3, NB, False)(
        ei32, ew32, p3)
    out = _tc_final(NP, F3, C, BLK)(acc3, p3, dinv, b3r)
    return out[:N]


# speedup vs baseline: 1.0068x; 1.0010x over previous
import functools

import jax
import jax.numpy as jnp
from jax import lax
from jax.experimental import pallas as pl
from jax.experimental.pallas import tpu as pltpu
from jax.experimental.pallas import tpu_sc as plsc

NC = 2
NS = 16
NW = NC * NS
L = 16
K = 128

_SC_PARAMS = pltpu.CompilerParams(
    needs_layout_passes=False, use_tc_tiling_on_sc=False)


def _sc_mesh():
    return plsc.VectorSubcoreMesh(core_axis_name="c", subcore_axis_name="s")


def _deg_kernel(NP, EP):
    T = EP // (NW * K)
    rpt = NP // NS

    @functools.partial(
        pl.kernel, mesh=_sc_mesh(),
        out_type=jax.ShapeDtypeStruct((NC, NP), jnp.float32),
        scratch_types=[
            pltpu.VMEM((T, 2, K), jnp.int32),
            pltpu.VMEM((T, K), jnp.float32),
            pltpu.VMEM((rpt,), jnp.float32),
            pltpu.VMEM_SHARED((NP,), jnp.float32),
        ],
        compiler_params=_SC_PARAMS,
    )
    def deg_k(ei_hbm, ew_hbm, out_hbm, ei_all, ew_all, zbuf, acc_sh):
        c = lax.axis_index("c")
        s = lax.axis_index("s")
        w = c * NS + s
        pltpu.sync_copy(ei_hbm.at[w], ei_all)
        pltpu.sync_copy(ew_hbm.at[w], ew_all)

        def zb(i, carry):
            zbuf[pl.ds(i * L, L)] = jnp.zeros((L,), jnp.float32)
            return carry
        lax.fori_loop(0, rpt // L, zb, 0)
        pltpu.sync_copy(zbuf, acc_sh.at[pl.ds(s * rpt, rpt)])
        plsc.subcore_barrier()

        def chunk(t, carry):
            pltpu.sync_copy(ew_all.at[t], acc_sh.at[ei_all.at[t, 1]], add=True)
            return carry
        lax.fori_loop(0, T, chunk, 0)
        plsc.subcore_barrier()
        pltpu.sync_copy(acc_sh.at[pl.ds(s * rpt, rpt)],
                        out_hbm.at[c, pl.ds(s * rpt, rpt)])

    return deg_k


def _pick_piece(T, F, NP, NB):
    budget = 2_000_000
    for P in range(T, 0, -1):
        if T % P or P % NB:
            continue
        if NS * (3 * P * K + NB * K * F) + NP * F <= budget:
            return P
    raise ValueError("no feasible staging piece size")


def _agg_kernel(NP, EP, F, feature_split, P, NB):
    ntile = NS if feature_split else NW
    T = EP // (ntile * K)
    QP = T // P
    assert T % P == 0 and P % NB == 0
    rpt = NP // NS
    ZC = rpt // K

    @functools.partial(
        pl.kernel, mesh=_sc_mesh(),
        out_type=jax.ShapeDtypeStruct((NC, NP, F), jnp.float32),
        scratch_types=[
            pltpu.VMEM((P, 2, K), jnp.int32),
            pltpu.VMEM((P, K), jnp.float32),
        ] + [pltpu.VMEM((K, F), jnp.float32) for _ in range(NB)]
          + [pltpu.VMEM_SHARED((NP, F), jnp.float32)]
          + [pltpu.SemaphoreType.DMA for _ in range(2 * NB)],
        compiler_params=_SC_PARAMS,
    )
    def agg_k(ei_hbm, ew_hbm, p_hbm, out_hbm,
              ei_all, ew_all, *bufs_acc_sems):
        rows = bufs_acc_sems[:NB]
        acc_sh = bufs_acc_sems[NB]
        gsem = bufs_acc_sems[NB + 1:2 * NB + 1]
        ssem = bufs_acc_sems[2 * NB + 1:]
        c = lax.axis_index("c")
        s = lax.axis_index("s")
        w = s if feature_split else c * NS + s

        def zb(i, carry):
            for j in range(F // L):
                rows[0][i, pl.ds(j * L, L)] = jnp.zeros((L,), jnp.float32)
            return carry
        lax.fori_loop(0, K, zb, 0)
        for z in range(ZC):
            pltpu.sync_copy(rows[0], acc_sh.at[pl.ds(s * rpt + z * K, K)])
        plsc.subcore_barrier()

        def scale(b, t):
            tv = jnp.full((L,), t, jnp.int32)

            def sbody(i, carry2):
                wv = plsc.load_gather(
                    ew_all, [tv, jnp.full((L,), i, jnp.int32)])
                for j in range(F // L):
                    sl = (i, pl.ds(j * L, L))
                    rows[b][sl] = rows[b][sl] * wv
                return carry2

            @plsc.parallel_loop(0, K, unroll=8)
            def _(i):
                sbody(i, 0)

        def gstart(b, t):
            pltpu.async_copy(p_hbm.at[ei_all.at[t, 0]], rows[b], gsem[b])

        def gwait(b):
            pltpu.make_async_copy(
                p_hbm.at[ei_all.at[0, 0]], rows[b], gsem[b]).wait()

        def sstart(b, t):
            pltpu.async_copy(
                rows[b], acc_sh.at[ei_all.at[t, 1]], ssem[b], add=True)

        def swait(b):
            pltpu.make_async_copy(
                rows[b], acc_sh.at[ei_all.at[0, 1]], ssem[b]).wait()

        def piece(q, carry):
            pltpu.sync_copy(ei_hbm.at[w, pl.ds(q * P, P)], ei_all)
            pltpu.sync_copy(ew_hbm.at[w, pl.ds(q * P, P)], ew_all)
            if feature_split:
                offv = jnp.full((L,), c * NP, jnp.int32)

                def ob(i, carry2):
                    for j in range(K // L):
                        sl = (i, 0, pl.ds(j * L, L))
                        ei_all[sl] = ei_all[sl] + offv
                    return carry2
                lax.fori_loop(0, P, ob, 0, unroll=4)

            for b in range(NB - 1):
                gstart(b, b)

            def rnd(r, carry2):
                for b in range(NB):
                    u = r * NB + b
                    gwait(b)
                    bp = (b - 1) % NB
                    scale(b, u)

                    @pl.when(u > 0)
                    def _():
                        swait(bp)
                    gstart(bp, lax.rem(u + NB - 1, P))
                    sstart(b, u)
                return carry2
            lax.fori_loop(0, P // NB, rnd, 0)
            for b in range(NB - 1):
                gwait(b)
            swait(NB - 1)
            return carry
        lax.fori_loop(0, QP, piece, 0)
        plsc.subcore_barrier()
        for z in range(ZC):
            sl = pl.ds(s * rpt + z * K, K)
            pltpu.sync_copy(acc_sh.at[sl], out_hbm.at[c, sl])

    return agg_k


def _tc_norm_first(NP, D, H1, BLK):
    grid = NP // BLK
    FH = H1 // 2

    def body(dega, degb, x_ref, w_ref, dinv_ref, p_ref):
        d = dega[...] + degb[...] + 2.0
        di = jnp.where(d > 0, lax.rsqrt(d), 0.0)
        g = jnp.dot(x_ref[...], w_ref[...], preferred_element_type=jnp.float32)
        dinv_ref[...] = di
        p = di * g
        p_ref[0] = p[:, :FH]
        p_ref[1] = p[:, FH:]

    return pl.pallas_call(
        body,
        grid=(grid,),
        in_specs=[
            pl.BlockSpec((BLK, 1), lambda i: (i, 0)),
            pl.BlockSpec((BLK, 1), lambda i: (i, 0)),
            pl.BlockSpec((BLK, D), lambda i: (i, 0)),
            pl.BlockSpec((D, H1), lambda i: (0, 0)),
        ],
        out_specs=[
            pl.BlockSpec((BLK, 1), lambda i: (i, 0)),
            pl.BlockSpec((2, BLK, FH), lambda i: (0, i, 0)),
        ],
        out_shape=[
            jax.ShapeDtypeStruct((NP, 1), jnp.float32),
            jax.ShapeDtypeStruct((2, NP, FH), jnp.float32),
        ],
    )


def _tc_mid(NP, Fin, Fout, BLK, out_split):
    grid = NP // BLK
    FH = Fout // 2

    def body(acc_ref, p_ref, dinv_ref, b_ref, w_ref, pout_ref):
        di = dinv_ref[...]
        agg = jnp.concatenate([acc_ref[0], acc_ref[1]], axis=1)
        pin = jnp.concatenate([p_ref[0], p_ref[1]], axis=1)
        a = di * agg + (2.0 * di) * pin + b_ref[...]
        h = jnp.maximum(a, 0.0)
        gn = jnp.dot(h, w_ref[...], preferred_element_type=jnp.float32)
        p = di * gn
        if out_split:
            pout_ref[0] = p[:, :FH]
            pout_ref[1] = p[:, FH:]
        else:
            pout_ref[...] = p

    return pl.pallas_call(
        body,
        grid=(grid,),
        in_specs=[
            pl.BlockSpec((2, BLK, Fin // 2), lambda i: (0, i, 0)),
            pl.BlockSpec((2, BLK, Fin // 2), lambda i: (0, i, 0)),
            pl.BlockSpec((BLK, 1), lambda i: (i, 0)),
            pl.BlockSpec((1, Fin), lambda i: (0, 0)),
            pl.BlockSpec((Fin, Fout), lambda i: (0, 0)),
        ],
        out_specs=(
            pl.BlockSpec((2, BLK, FH), lambda i: (0, i, 0)) if out_split
            else pl.BlockSpec((BLK, Fout), lambda i: (i, 0))),
        out_shape=(
            jax.ShapeDtypeStruct((2, NP, FH), jnp.float32) if out_split
            else jax.ShapeDtypeStruct((NP, Fout), jnp.float32)),
    )


def _tc_final(NP, F3, C, BLK):
    grid = NP // BLK

    def body(acc_ref, p_ref, dinv_ref, b_ref, out_ref):
        di = dinv_ref[...]
        a = (di * (acc_ref[0] + acc_ref[1])
             + (2.0 * di) * p_ref[...] + b_ref[...])
        mask = lax.broadcasted_iota(jnp.int32, (BLK, F3), 1) < C
        am = jnp.where(mask, a, -1e30)
        m = jnp.max(am, axis=1, keepdims=True)
        z = am - m
        e = jnp.where(mask, jnp.exp(z), 0.0)
        ssum = jnp.sum(e, axis=1, keepdims=True)
        out_ref[...] = (z - jnp.log(ssum))[:, :C]

    return pl.pallas_call(
        body,
        grid=(grid,),
        in_specs=[
            pl.BlockSpec((2, BLK, F3), lambda i: (0, i, 0)),
            pl.BlockSpec((BLK, F3), lambda i: (i, 0)),
            pl.BlockSpec((BLK, 1), lambda i: (i, 0)),
            pl.BlockSpec((1, F3), lambda i: (0, 0)),
        ],
        out_specs=pl.BlockSpec((BLK, C), lambda i: (i, 0)),
        out_shape=jax.ShapeDtypeStruct((NP, C), jnp.float32),
    )


def kernel(x, edge_index, edge_attr, W1, b1, W2, b2, W3, b3):
    N, D = x.shape
    E = edge_index.shape[1]
    H1 = W1.shape[1]
    H2 = W2.shape[1]
    C = W3.shape[1]
    F3 = 16
    NP = -(-N // 2048) * 2048
    EP = -(-E // (NW * K * 2)) * (NW * K * 2)
    BLK = 2048

    if E % K:
        ek = -(-E // K) * K - E
        fill = (jnp.arange(ek, dtype=jnp.int32) % N).astype(jnp.int32)
        edge_index = jnp.concatenate(
            [edge_index, jnp.broadcast_to(fill, (2, ek))], axis=1)
        edge_attr = jnp.concatenate(
            [edge_attr, jnp.zeros((ek,), edge_attr.dtype)])
        E += ek
    ei3 = jnp.transpose(edge_index.reshape(2, E // K, K), (1, 0, 2))
    pad = EP - E
    if pad:
        pidx = (jnp.arange(pad, dtype=jnp.int32) % N).astype(jnp.int32)
        pad3 = jnp.broadcast_to(
            pidx.reshape(pad // K, 1, K), (pad // K, 2, K))
        ei3 = jnp.concatenate([ei3, pad3], axis=0)
        ew = jnp.concatenate([edge_attr, jnp.zeros((pad,), edge_attr.dtype)])
    else:
        ew = edge_attr
    ei32 = ei3.reshape(NW, -1, 2, K)
    ew32 = ew.reshape(NW, -1, K)
    ei16 = ei3.reshape(NS, -1, 2, K)
    ew16 = ew.reshape(NS, -1, K)
    xp = jnp.pad(x, ((0, NP - N), (0, 0)))
    W3p = jnp.pad(W3, ((0, 0), (0, F3 - C)))
    b1r = b1[None, :]
    b2r = b2[None, :]
    b3r = jnp.pad(b3, (0, F3 - C))[None, :]

    deg = _deg_kernel(NP, EP)(ei32, ew32)
    dega = deg[0][:, None]
    degb = deg[1][:, None]
    dinv, p1s = _tc_norm_first(NP, D, H1, BLK)(dega, degb, xp, W1)
    T16 = EP // (NS * K)
    T32 = EP // (NW * K)
    NB = 4 if T32 % 4 == 0 else 2
    p1f = p1s.reshape(2 * NP, H1 // 2)
    P1 = _pick_piece(T16, H1 // 2, NP, NB)
    acc1 = _agg_kernel(NP, EP, H1 // 2, True, P1, NB)(
        ei16, ew16, p1f)
    p2s = _tc_mid(NP, H1, H2, BLK, True)(acc1, p1s, dinv, b1r, W2)
    p2f = p2s.reshape(2 * NP, H2 // 2)
    P2 = _pick_piece(T16, H2 // 2, NP, NB)
    acc2 = _agg_kernel(NP, EP, H2 // 2, True, P2, NB)(
        ei16, ew16, p2f)
    p3 = _tc_mid(NP, H2, F3, BLK, False)(acc2, p2s, dinv, b2r, W3p)
    P3 = _pick_piece(T32, F3, NP, NB)
    acc3 = _agg_kernel(NP, EP, F3, False, P3, NB)(
        ei32, ew32, p3)
    out = _tc_final(NP, F3, C, BLK)(acc3, p3, dinv, b3r)
    return out[:N]
